# baseline (speedup 1.0000x reference)
import functools

import jax
import jax.numpy as jnp
from jax import lax
from jax.experimental import pallas as pl
from jax.experimental.pallas import tpu as pltpu
from jax.experimental.pallas import tpu_sc as plsc

F = 26
V = 100000
D = 64
B = 16384

NC = 2
NS = 16
NW = NC * NS
TILE = 128
ROWS = (F * B) // TILE
NT = ROWS // NW
NB = 8


@functools.partial(
    pl.kernel,
    out_type=jax.ShapeDtypeStruct((ROWS, TILE, D), jnp.float32),
    mesh=plsc.VectorSubcoreMesh(core_axis_name="c", subcore_axis_name="s"),
    compiler_params=pltpu.CompilerParams(use_tc_tiling_on_sc=False),
    scratch_types=(
        [pltpu.VMEM((NT, TILE), jnp.int32)]
        + [pltpu.VMEM((TILE, D), jnp.float32) for _ in range(NB)]
        + [pltpu.SemaphoreType.DMA for _ in range(2 * NB)]
    ),
)
def _emb_lookup(table_hbm, idx_hbm, out_hbm, idx_v, *rest):
    bufs = rest[:NB]
    gsems = rest[NB:2 * NB]
    osems = rest[2 * NB:3 * NB]

    wid = lax.axis_index("s") * NC + lax.axis_index("c")
    row0 = wid * NT

    pltpu.sync_copy(idx_hbm.at[pl.ds(row0, NT)], idx_v)

    def adjust(r, carry):
        off = ((row0 + r) >> 7) * V
        for j in range(TILE // 16):
            sl = pl.ds(j * 16, 16)
            idx_v[r, sl] = idx_v[r, sl] + off
        return carry

    lax.fori_loop(0, NT, adjust, 0)

    def g_start(t, b):
        pltpu.async_copy(table_hbm.at[idx_v.at[t]], bufs[b], gsems[b])

    def g_wait(t, b):
        pltpu.make_async_copy(table_hbm.at[idx_v.at[t]], bufs[b], gsems[b]).wait()

    def o_start(t, b):
        pltpu.async_copy(bufs[b], out_hbm.at[row0 + t], osems[b])

    def o_wait(t, b):
        pltpu.make_async_copy(bufs[b], out_hbm.at[row0 + t], osems[b]).wait()

    for b in range(NB):
        g_start(b, b)

    def outer(k, carry):
        g = k * NB
        for b in range(NB):
            t = g + b
            g_wait(t, b)
            o_start(t, b)
            o_wait(t, b)
            nxt = t + NB

            @pl.when(nxt < NT)
            def _():
                g_start(nxt, b)
        return carry

    lax.fori_loop(0, NT // NB, outer, 0)


def kernel(x, tables):
    xt = x.astype(jnp.int32).T.reshape(ROWS, TILE)
    tab = tables.reshape(F * V, D)
    out = _emb_lookup(tab, xt)
    return out.reshape(F, B, D)


# speedup vs baseline: 1.0010x; 1.0010x over previous
import functools

import jax
import jax.numpy as jnp
from jax import lax
from jax.experimental import pallas as pl
from jax.experimental.pallas import tpu as pltpu
from jax.experimental.pallas import tpu_sc as plsc

F = 26
V = 100000
D = 64
B = 16384

NC = 2
NS = 16
NW = NC * NS
TILE = 128
ROWS = (F * B) // TILE
NT = ROWS // NW
NB = 8


@functools.partial(
    pl.kernel,
    out_type=jax.ShapeDtypeStruct((F, B, D), jnp.float32),
    mesh=plsc.VectorSubcoreMesh(core_axis_name="c", subcore_axis_name="s"),
    compiler_params=pltpu.CompilerParams(use_tc_tiling_on_sc=False),
    scratch_types=(
        [pltpu.VMEM((NT, TILE), jnp.int32)]
        + [pltpu.VMEM((TILE, D), jnp.float32) for _ in range(NB)]
        + [pltpu.SemaphoreType.DMA for _ in range(2 * NB)]
    ),
)
def _emb_lookup(table_hbm, idx_hbm, out_hbm, idx_v, *rest):
    bufs = rest[:NB]
    gsems = rest[NB:2 * NB]
    osems = rest[2 * NB:3 * NB]

    wid = lax.axis_index("s") * NC + lax.axis_index("c")
    row0 = wid * NT

    pltpu.sync_copy(idx_hbm.at[pl.ds(row0, NT)], idx_v)

    def g_start(t, b):
        f = (row0 + t) >> 7
        src = table_hbm.at[f].at[idx_v.at[t]]
        pltpu.async_copy(src, bufs[b], gsems[b])

    def g_wait(t, b):
        f = (row0 + t) >> 7
        src = table_hbm.at[f].at[idx_v.at[t]]
        pltpu.make_async_copy(src, bufs[b], gsems[b]).wait()

    def _dst(t):
        g = row0 + t
        return out_hbm.at[g >> 7].at[pl.ds((g & 127) * TILE, TILE)]

    def o_start(t, b):
        pltpu.async_copy(bufs[b], _dst(t), osems[b])

    def o_wait(t, b):
        pltpu.make_async_copy(bufs[b], _dst(t), osems[b]).wait()

    for b in range(NB):
        g_start(b, b)

    def outer(k, carry):
        g = k * NB
        for b in range(NB):
            t = g + b
            g_wait(t, b)
            o_start(t, b)
            o_wait(t, b)
            nxt = t + NB

            @pl.when(nxt < NT)
            def _():
                g_start(nxt, b)
        return carry

    lax.fori_loop(0, NT // NB, outer, 0)


def kernel(x, tables):
    xt = x.astype(jnp.int32).T.reshape(ROWS, TILE)
    return _emb_lookup(tables, xt)
